# baseline (speedup 1.0000x reference)
import functools

import jax
import jax.numpy as jnp
from jax import lax
from jax.experimental import pallas as pl
from jax.experimental.pallas import tpu as pltpu
from jax.experimental.pallas import tpu_sc as plsc

N = 10000
E = 320000
NFEAT = 128
NHID = 256
HH = 128
NOUT = 32

NC = 2
NS = 16
EPW = E // NS
K = 80
NB = EPW // K
SLAB = 624
REM = N - NS * SLAB

RB = 1000
GRID = N // RB


NBUF = 3
CH = 10
NCH = NB // CH
MAIN = (NB - 1) // NBUF * NBUF


def _spmm_body(pk_h, w_h, sup_h, z_h, out_h, pk_v, w_v, bufs, gsems,
               ssems, pksem, wsem, acc_s):
    c = lax.axis_index("c")
    s = lax.axis_index("s")

    soff = pl.multiple_of(s * SLAB, 8)
    pltpu.sync_copy(z_h.at[pl.ds(soff, SLAB)], acc_s.at[pl.ds(soff, SLAB)])

    @pl.when(s == 0)
    def _():
        pltpu.sync_copy(z_h.at[pl.ds(NS * SLAB, REM)],
                        acc_s.at[pl.ds(NS * SLAB, REM)])

    pltpu.sync_copy(pk_h.at[c, s, 0], pk_v.at[0])
    pltpu.sync_copy(w_h.at[s, 0], w_v.at[0])
    plsc.subcore_barrier()

    def gather(b, j):
        return pltpu.async_copy(
            sup_h.at[pk_v.at[(b // CH) % 2, b % CH, 0]], bufs[j], gsems[j])

    def wait_gather(b, j):
        pltpu.make_async_copy(
            sup_h.at[pk_v.at[(b // CH) % 2, b % CH, 0]], bufs[j],
            gsems[j]).wait()

    def scatter(b, j):
        return pltpu.async_copy(
            bufs[j], acc_s.at[pk_v.at[(b // CH) % 2, b % CH, 1]], ssems[j],
            add=True)

    def wait_scatter(b, j):
        pltpu.make_async_copy(
            bufs[j], acc_s.at[pk_v.at[(b // CH) % 2, b % CH, 1]],
            ssems[j]).wait()

    def scale(b, j):
        d = (b // CH) % 2
        bb = b % CH

        def group(g, carry2):
            goff = pl.multiple_of(g * 16, 16)
            w16 = w_v[d, bb, pl.ds(goff, 16)]
            for u in range(16):
                we = w16[u]
                e = goff + u
                for k in range(HH // 16):
                    bufs[j][e, pl.ds(k * 16, 16)] = (
                        bufs[j][e, pl.ds(k * 16, 16)] * we)
            return carry2
        lax.fori_loop(0, K // 16, group, 0, unroll=K // 16)

    gather(0, 0)
    gather(1, 1)

    def step(o, carry):
        for j in range(NBUF):
            b = o * NBUF + j
            wait_gather(b, j)
            scale(b, j)
            scatter(b, j)

            bn = b + 2
            jn = (j + 2) % NBUF
            n = b // CH
            bb = b % CH

            @pl.when(jnp.logical_and(bb == CH - 2, n < NCH - 1))
            def _():
                pltpu.make_async_copy(
                    pk_h.at[c, s, n + 1],
                    pk_v.at[(n + 1) % 2], pksem).wait()
                pltpu.make_async_copy(
                    w_h.at[s, n + 1],
                    w_v.at[(n + 1) % 2], wsem).wait()

            @pl.when(bn < NB)
            def _():
                @pl.when(bn >= NBUF)
                def _():
                    wait_scatter(bn - NBUF, jn)
                gather(bn, jn)

            @pl.when(jnp.logical_and(bb == 0, n + 1 < NCH))
            def _():
                pltpu.async_copy(pk_h.at[c, s, n + 1],
                                 pk_v.at[(n + 1) % 2], pksem)
                pltpu.async_copy(w_h.at[s, n + 1],
                                 w_v.at[(n + 1) % 2], wsem)
        return carry

    lax.fori_loop(0, MAIN // NBUF, step, 0)

    for b in range(MAIN, NB):
        j = b % NBUF
        wait_gather(b, j)
        scale(b, j)
        scatter(b, j)

    for b in range(NB - NBUF, NB):
        wait_scatter(b, b % NBUF)

    plsc.subcore_barrier()
    pl.delay(2000)
    plsc.subcore_barrier()

    pltpu.sync_copy(acc_s.at[pl.ds(soff, SLAB)],
                    out_h.at[c, pl.ds(soff, SLAB)])

    @pl.when(s == 0)
    def _():
        pltpu.sync_copy(acc_s.at[pl.ds(NS * SLAB, REM)],
                        out_h.at[c, pl.ds(NS * SLAB, REM)])


@functools.cache
def _spmm_kernel():
    return pl.kernel(
        _spmm_body,
        out_type=jax.ShapeDtypeStruct((NC, N, HH), jnp.float32),
        mesh=plsc.VectorSubcoreMesh(core_axis_name="c", subcore_axis_name="s",
                                    num_cores=NC, num_subcores=NS),
        scratch_types=[
            pltpu.VMEM((2, CH, 2, K), jnp.int32),
            pltpu.VMEM((2, CH, K), jnp.float32),
            tuple(pltpu.VMEM((K, HH), jnp.float32) for _ in range(NBUF)),
            tuple(pltpu.SemaphoreType.DMA for _ in range(NBUF)),
            tuple(pltpu.SemaphoreType.DMA for _ in range(NBUF)),
            pltpu.SemaphoreType.DMA,
            pltpu.SemaphoreType.DMA,
            pltpu.VMEM_SHARED((N, HH), jnp.float32),
        ],
    )


def _spmm_call(col, row, w, sup, zeros):
    col2 = jnp.stack([col, col + N]).reshape(NC, NS, NCH, CH, 1, K)
    row4 = jnp.broadcast_to(row.reshape(1, NS, NCH, CH, 1, K),
                            (NC, NS, NCH, CH, 1, K))
    pk = jnp.concatenate([col2, row4], axis=4)
    w3 = w.reshape(NS, NCH, CH, K)
    return _spmm_kernel()(pk, w3, sup, zeros)


def _inproj_body(x_ref, w_ref, oe_ref, os_ref):
    y = jnp.dot(x_ref[...], w_ref[...], preferred_element_type=jnp.float32)
    oe_ref[0] = y[:, 0:HH]
    oe_ref[1] = y[:, HH:2 * HH]
    os_ref[0] = y[:, 2 * HH:3 * HH]
    os_ref[1] = y[:, 3 * HH:4 * HH]


def _inproj(x, wcat):
    return pl.pallas_call(
        _inproj_body,
        grid=(GRID,),
        in_specs=[
            pl.BlockSpec((RB, NFEAT), lambda r: (r, 0)),
            pl.BlockSpec((NFEAT, 2 * NHID), lambda r: (0, 0)),
        ],
        out_specs=[
            pl.BlockSpec((NC, RB, HH), lambda r: (0, r, 0)),
            pl.BlockSpec((NC, RB, HH), lambda r: (0, r, 0)),
        ],
        out_shape=[
            jax.ShapeDtypeStruct((NC, N, HH), jnp.float32),
            jax.ShapeDtypeStruct((NC, N, HH), jnp.float32),
        ],
    )(x, wcat)


def _stats2_body(ue_ref, us_ref, ge_ref, be_ref, gs_ref, bs_ref,
                 ste_ref, sts_ref, acc_ref):
    r = pl.program_id(0)

    @pl.when(r == 0)
    def _():
        acc_ref[...] = jnp.zeros_like(acc_ref)

    ue = ue_ref[...]
    us = us_ref[...]
    for half in range(NC):
        sl = slice(half * HH, (half + 1) * HH)
        acc_ref[0:1, sl] += jnp.sum(ue[half], axis=0, keepdims=True)
        acc_ref[1:2, sl] += jnp.sum(ue[half] * ue[half], axis=0, keepdims=True)
        acc_ref[2:3, sl] += jnp.sum(us[half], axis=0, keepdims=True)
        acc_ref[3:4, sl] += jnp.sum(us[half] * us[half], axis=0, keepdims=True)

    @pl.when(r == GRID - 1)
    def _():
        inv_n = 1.0 / N
        for (row0, g_ref, b_ref, out_ref) in ((0, ge_ref, be_ref, ste_ref),
                                              (2, gs_ref, bs_ref, sts_ref)):
            m = acc_ref[row0:row0 + 1, :] * inv_n
            v = acc_ref[row0 + 1:row0 + 2, :] * inv_n - m * m
            a = g_ref[...] * lax.rsqrt(v + 1e-5)
            b = b_ref[...] - a * m
            out_ref[...] = jnp.concatenate(
                [a, b, jnp.zeros((6, NHID), jnp.float32)], axis=0)


def _stats2(ue, us, ge, be, gs, bs):
    return pl.pallas_call(
        _stats2_body,
        grid=(GRID,),
        in_specs=[
            pl.BlockSpec((NC, RB, HH), lambda r: (0, r, 0)),
            pl.BlockSpec((NC, RB, HH), lambda r: (0, r, 0)),
            pl.BlockSpec((1, NHID), lambda r: (0, 0)),
            pl.BlockSpec((1, NHID), lambda r: (0, 0)),
            pl.BlockSpec((1, NHID), lambda r: (0, 0)),
            pl.BlockSpec((1, NHID), lambda r: (0, 0)),
        ],
        out_specs=[
            pl.BlockSpec((8, NHID), lambda r: (0, 0)),
            pl.BlockSpec((8, NHID), lambda r: (0, 0)),
        ],
        out_shape=[
            jax.ShapeDtypeStruct((8, NHID), jnp.float32),
            jax.ShapeDtypeStruct((8, NHID), jnp.float32),
        ],
        scratch_shapes=[pltpu.VMEM((8, NHID), jnp.float32)],
    )(ue, us, ge, be, gs, bs)


def _act_half(u, st, half):
    sl = slice(half * HH, (half + 1) * HH)
    hn = u * st[0:1, sl] + st[1:2, sl]
    return jnp.where(hn > 0, hn, jnp.exp(hn) - 1.0)


def _apply2_body(ue_ref, us_ref, ste_ref, sts_ref, we_ref, ws_ref,
                 oe_ref, os_ref):
    ste = ste_ref[...]
    sts = sts_ref[...]
    for (u_ref, st, w_ref, o_ref) in ((ue_ref, ste, we_ref, oe_ref),
                                      (us_ref, sts, ws_ref, os_ref)):
        u = u_ref[...]
        y = jnp.dot(_act_half(u[0], st, 0), w_ref[0:HH, :],
                    preferred_element_type=jnp.float32)
        y += jnp.dot(_act_half(u[1], st, 1), w_ref[HH:2 * HH, :],
                     preferred_element_type=jnp.float32)
        o_ref[0] = y[:, 0:HH]
        o_ref[1] = y[:, HH:2 * HH]


def _apply2(ue, us, ste, sts, we, ws):
    return pl.pallas_call(
        _apply2_body,
        grid=(GRID,),
        in_specs=[
            pl.BlockSpec((NC, RB, HH), lambda r: (0, r, 0)),
            pl.BlockSpec((NC, RB, HH), lambda r: (0, r, 0)),
            pl.BlockSpec((8, NHID), lambda r: (0, 0)),
            pl.BlockSpec((8, NHID), lambda r: (0, 0)),
            pl.BlockSpec((NHID, NHID), lambda r: (0, 0)),
            pl.BlockSpec((NHID, NHID), lambda r: (0, 0)),
        ],
        out_specs=[
            pl.BlockSpec((NC, RB, HH), lambda r: (0, r, 0)),
            pl.BlockSpec((NC, RB, HH), lambda r: (0, r, 0)),
        ],
        out_shape=[
            jax.ShapeDtypeStruct((NC, N, HH), jnp.float32),
            jax.ShapeDtypeStruct((NC, N, HH), jnp.float32),
        ],
    )(ue, us, ste, sts, we, ws)


def _concat_body(ue_ref, us_ref, ste_ref, sts_ref, w_ref, o_ref):
    ste = ste_ref[...]
    sts = sts_ref[...]
    ue = ue_ref[...]
    us = us_ref[...]
    w = w_ref[...]
    y = jnp.dot(_act_half(ue[0], ste, 0), w[0:HH, :],
                preferred_element_type=jnp.float32)
    y += jnp.dot(_act_half(ue[1], ste, 1), w[HH:2 * HH, :],
                 preferred_element_type=jnp.float32)
    y += jnp.dot(_act_half(us[0], sts, 0), w[2 * HH:3 * HH, :],
                 preferred_element_type=jnp.float32)
    y += jnp.dot(_act_half(us[1], sts, 1), w[3 * HH:4 * HH, :],
                 preferred_element_type=jnp.float32)
    o_ref[...] = y


def _concat_mm(ue, us, ste, sts, w):
    return pl.pallas_call(
        _concat_body,
        grid=(GRID,),
        in_specs=[
            pl.BlockSpec((NC, RB, HH), lambda r: (0, r, 0)),
            pl.BlockSpec((NC, RB, HH), lambda r: (0, r, 0)),
            pl.BlockSpec((8, NHID), lambda r: (0, 0)),
            pl.BlockSpec((8, NHID), lambda r: (0, 0)),
            pl.BlockSpec((2 * NHID, NHID), lambda r: (0, 0)),
        ],
        out_specs=pl.BlockSpec((RB, NHID), lambda r: (r, 0)),
        out_shape=jax.ShapeDtypeStruct((N, NHID), jnp.float32),
    )(ue, us, ste, sts, w)


def _statsf_body(h_ref, g_ref, b_ref, st_ref, acc_ref):
    r = pl.program_id(0)

    @pl.when(r == 0)
    def _():
        acc_ref[...] = jnp.zeros_like(acc_ref)

    h = h_ref[...]
    acc_ref[0:1, :] += jnp.sum(h, axis=0, keepdims=True)
    acc_ref[1:2, :] += jnp.sum(h * h, axis=0, keepdims=True)

    @pl.when(r == GRID - 1)
    def _():
        inv_n = 1.0 / N
        m = acc_ref[0:1, :] * inv_n
        v = acc_ref[1:2, :] * inv_n - m * m
        a = g_ref[...] * lax.rsqrt(v + 1e-5)
        b = b_ref[...] - a * m
        st_ref[...] = jnp.concatenate(
            [a, b, jnp.zeros((6, NHID), jnp.float32)], axis=0)


def _statsf(h, g, b):
    return pl.pallas_call(
        _statsf_body,
        grid=(GRID,),
        in_specs=[
            pl.BlockSpec((RB, NHID), lambda r: (r, 0)),
            pl.BlockSpec((1, NHID), lambda r: (0, 0)),
            pl.BlockSpec((1, NHID), lambda r: (0, 0)),
        ],
        out_specs=pl.BlockSpec((8, NHID), lambda r: (0, 0)),
        out_shape=jax.ShapeDtypeStruct((8, NHID), jnp.float32),
        scratch_shapes=[pltpu.VMEM((8, NHID), jnp.float32)],
    )(h, g, b)


def _applyf_body(h_ref, st_ref, w_ref, o_ref):
    st = st_ref[...]
    hn = h_ref[...] * st[0:1, :] + st[1:2, :]
    hact = jnp.where(hn > 0, hn, jnp.exp(hn) - 1.0)
    o_ref[...] = jnp.dot(hact, w_ref[...], preferred_element_type=jnp.float32)


def _applyf(h, st, w):
    return pl.pallas_call(
        _applyf_body,
        grid=(GRID,),
        in_specs=[
            pl.BlockSpec((RB, NHID), lambda r: (r, 0)),
            pl.BlockSpec((8, NHID), lambda r: (0, 0)),
            pl.BlockSpec((NHID, NHID), lambda r: (0, 0)),
        ],
        out_specs=pl.BlockSpec((RB, NHID), lambda r: (r, 0)),
        out_shape=jax.ShapeDtypeStruct((N, NHID), jnp.float32),
    )(h, st, w)


def _final_body(h_ref, st_ref, w_ref, bias_ref, o_ref):
    st = st_ref[...]
    hn = h_ref[...] * st[0:1, :] + st[1:2, :]
    hact = jnp.where(hn > 0, hn, jnp.exp(hn) - 1.0)
    z = jnp.dot(hact, w_ref[...], preferred_element_type=jnp.float32)
    z += bias_ref[...]
    m = jnp.max(z, axis=1, keepdims=True)
    zs = z - m
    lse = jnp.log(jnp.sum(jnp.exp(zs), axis=1, keepdims=True))
    o_ref[...] = zs - lse


def _final(h, st, w, bias):
    return pl.pallas_call(
        _final_body,
        grid=(GRID,),
        in_specs=[
            pl.BlockSpec((RB, NHID), lambda r: (r, 0)),
            pl.BlockSpec((8, NHID), lambda r: (0, 0)),
            pl.BlockSpec((NHID, NOUT), lambda r: (0, 0)),
            pl.BlockSpec((1, NOUT), lambda r: (0, 0)),
        ],
        out_specs=pl.BlockSpec((RB, NOUT), lambda r: (r, 0)),
        out_shape=jax.ShapeDtypeStruct((N, NOUT), jnp.float32),
    )(h, st, w, bias)


def kernel(x, edge_index_exp, edge_weight_exp, edge_index_sp, edge_weight_sp,
           params):
    p = params
    row_e, col_e = edge_index_exp[0], edge_index_exp[1]
    row_s, col_s = edge_index_sp[0], edge_index_sp[1]
    zeros_hbm = jnp.zeros((N, HH), jnp.float32)

    def spmm(row, col, w, sup):
        return _spmm_call(col, row, w, sup.reshape(NC * N, HH), zeros_hbm)

    def r2(v):
        return v.reshape(1, -1)

    wcat = jnp.concatenate([p['W_in_exp'], p['W_in_sp']], axis=1)
    se, ss = _inproj(x, wcat)

    for i in (0, 1, 2):
        sfx = ('in', 'c1', 'c2')[i]
        ue = spmm(row_e, col_e, edge_weight_exp, se)
        us = spmm(row_s, col_s, edge_weight_sp, ss)
        ste, sts = _stats2(ue, us,
                           r2(p['g_%s_exp' % sfx]), r2(p['beta_%s_exp' % sfx]),
                           r2(p['g_%s_sp' % sfx]), r2(p['beta_%s_sp' % sfx]))
        if i < 2:
            nxt = ('c1', 'c2')[i]
            se, ss = _apply2(ue, us, ste, sts,
                             p['W_%s_exp' % nxt], p['W_%s_sp' % nxt])
        else:
            o1 = _concat_mm(ue, us, ste, sts, p['W_out11'])

    st1 = _statsf(o1, r2(p['g_out1']), r2(p['beta_out1']))
    o2 = _applyf(o1, st1, p['W_out111'])
    st2 = _statsf(o2, r2(p['g_out111']), r2(p['beta_out111']))
    return _final(o2, st2, p['W_out12'], r2(p['b_out12']))


# speedup vs baseline: 1.3113x; 1.3113x over previous
import functools

import jax
import jax.numpy as jnp
from jax import lax
from jax.experimental import pallas as pl
from jax.experimental.pallas import tpu as pltpu
from jax.experimental.pallas import tpu_sc as plsc

N = 10000
E = 320000
NFEAT = 128
NHID = 256
HH = 128
NOUT = 32

NC = 2
NS = 16
EPW = E // NS
K = 80
NB = EPW // K
SLAB = 624
REM = N - NS * SLAB

RB = 1000
GRID = N // RB


NBUF = 3
CH = 10
NCH = NB // CH
MAIN = (NB - 1) // NBUF * NBUF


def _spmm_body(pk_h, w_h, sup_h, z_h, out_h, pk_v, w_v, bufs, gsems,
               ssems, pksem, wsem, acc_s):
    c = lax.axis_index("c")
    s = lax.axis_index("s")
    soff = pl.multiple_of(s * SLAB, 8)

    def branch_body(br, carry0):
      pltpu.sync_copy(z_h.at[pl.ds(soff, SLAB)], acc_s.at[pl.ds(soff, SLAB)])

      @pl.when(s == 0)
      def _():
          pltpu.sync_copy(z_h.at[pl.ds(NS * SLAB, REM)],
                          acc_s.at[pl.ds(NS * SLAB, REM)])

      pltpu.sync_copy(pk_h.at[br, c, s, 0], pk_v.at[0])
      pltpu.sync_copy(w_h.at[br, s, 0], w_v.at[0])
      plsc.subcore_barrier()

      def gather(b, j):
          return pltpu.async_copy(
              sup_h.at[pk_v.at[(b // CH) % 2, b % CH, 0]], bufs[j], gsems[j])

      def wait_gather(b, j):
          pltpu.make_async_copy(
              sup_h.at[pk_v.at[(b // CH) % 2, b % CH, 0]], bufs[j],
              gsems[j]).wait()

      def scatter(b, j):
          return pltpu.async_copy(
              bufs[j], acc_s.at[pk_v.at[(b // CH) % 2, b % CH, 1]], ssems[j],
              add=True)

      def wait_scatter(b, j):
          pltpu.make_async_copy(
              bufs[j], acc_s.at[pk_v.at[(b // CH) % 2, b % CH, 1]],
              ssems[j]).wait()

      def scale(b, j):
          d = (b // CH) % 2
          bb = b % CH

          def group(g, carry2):
              goff = pl.multiple_of(g * 16, 16)
              w16 = w_v[d, bb, pl.ds(goff, 16)]
              for u in range(16):
                  we = w16[u]
                  e = goff + u
                  for k in range(HH // 16):
                      bufs[j][e, pl.ds(k * 16, 16)] = (
                          bufs[j][e, pl.ds(k * 16, 16)] * we)
              return carry2
          lax.fori_loop(0, K // 16, group, 0)

      gather(0, 0)
      gather(1, 1)

      def step(o, carry):
          for j in range(NBUF):
              b = o * NBUF + j
              wait_gather(b, j)
              scale(b, j)
              scatter(b, j)

              bn = b + 2
              jn = (j + 2) % NBUF
              n = b // CH
              bb = b % CH

              @pl.when(jnp.logical_and(bb == CH - 2, n < NCH - 1))
              def _():
                  pltpu.make_async_copy(
                      pk_h.at[br, c, s, n + 1],
                      pk_v.at[(n + 1) % 2], pksem).wait()
                  pltpu.make_async_copy(
                      w_h.at[br, s, n + 1],
                      w_v.at[(n + 1) % 2], wsem).wait()

              @pl.when(bn < NB)
              def _():
                  @pl.when(bn >= NBUF)
                  def _():
                      wait_scatter(bn - NBUF, jn)
                  gather(bn, jn)

              @pl.when(jnp.logical_and(bb == 0, n + 1 < NCH))
              def _():
                  pltpu.async_copy(pk_h.at[br, c, s, n + 1],
                                   pk_v.at[(n + 1) % 2], pksem)
                  pltpu.async_copy(w_h.at[br, s, n + 1],
                                   w_v.at[(n + 1) % 2], wsem)
          return carry

      lax.fori_loop(0, MAIN // NBUF, step, 0)

      for b in range(MAIN, NB):
          j = b % NBUF
          wait_gather(b, j)
          scale(b, j)
          scatter(b, j)

      for b in range(NB - NBUF, NB):
          wait_scatter(b, b % NBUF)

      plsc.subcore_barrier()
      pl.delay(2000)
      plsc.subcore_barrier()

      pltpu.sync_copy(acc_s.at[pl.ds(soff, SLAB)],
                      out_h.at[br, c, pl.ds(soff, SLAB)])

      @pl.when(s == 0)
      def _():
          pltpu.sync_copy(acc_s.at[pl.ds(NS * SLAB, REM)],
                          out_h.at[br, c, pl.ds(NS * SLAB, REM)])

      plsc.subcore_barrier()
      return carry0

    lax.fori_loop(0, 2, branch_body, 0)


@functools.cache
def _spmm_kernel():
    return pl.kernel(
        _spmm_body,
        out_type=jax.ShapeDtypeStruct((2, NC, N, HH), jnp.float32),
        mesh=plsc.VectorSubcoreMesh(core_axis_name="c", subcore_axis_name="s",
                                    num_cores=NC, num_subcores=NS),
        scratch_types=[
            pltpu.VMEM((2, CH, 2, K), jnp.int32),
            pltpu.VMEM((2, CH, K), jnp.float32),
            tuple(pltpu.VMEM((K, HH), jnp.float32) for _ in range(NBUF)),
            tuple(pltpu.SemaphoreType.DMA for _ in range(NBUF)),
            tuple(pltpu.SemaphoreType.DMA for _ in range(NBUF)),
            pltpu.SemaphoreType.DMA,
            pltpu.SemaphoreType.DMA,
            pltpu.VMEM_SHARED((N, HH), jnp.float32),
        ],
    )


def _spmm_call(col_e, row_e, w_e, col_s, row_s, w_s, sup, zeros):
    def pack(col, row, br):
        col2 = jnp.stack([col + (2 * br) * N, col + (2 * br + 1) * N])
        col2 = col2.reshape(NC, NS, NCH, CH, 1, K)
        row4 = jnp.broadcast_to(row.reshape(1, NS, NCH, CH, 1, K),
                                (NC, NS, NCH, CH, 1, K))
        return jnp.concatenate([col2, row4], axis=4)

    pk = jnp.stack([pack(col_e, row_e, 0), pack(col_s, row_s, 1)])
    wch = jnp.stack([w_e.reshape(NS, NCH, CH, K),
                     w_s.reshape(NS, NCH, CH, K)])
    return _spmm_kernel()(pk, wch, sup.reshape(2 * NC * N, HH), zeros)


def _inproj_body(x_ref, w_ref, o_ref):
    y = jnp.dot(x_ref[...], w_ref[...], preferred_element_type=jnp.float32)
    o_ref[0, 0] = y[:, 0:HH]
    o_ref[0, 1] = y[:, HH:2 * HH]
    o_ref[1, 0] = y[:, 2 * HH:3 * HH]
    o_ref[1, 1] = y[:, 3 * HH:4 * HH]


def _inproj(x, wcat):
    return pl.pallas_call(
        _inproj_body,
        grid=(GRID,),
        in_specs=[
            pl.BlockSpec((RB, NFEAT), lambda r: (r, 0)),
            pl.BlockSpec((NFEAT, 2 * NHID), lambda r: (0, 0)),
        ],
        out_specs=pl.BlockSpec((2, NC, RB, HH), lambda r: (0, 0, r, 0)),
        out_shape=jax.ShapeDtypeStruct((2, NC, N, HH), jnp.float32),
    )(x, wcat)


def _stats2_body(u_ref, ge_ref, be_ref, gs_ref, bs_ref,
                 ste_ref, sts_ref, acc_ref):
    r = pl.program_id(0)

    @pl.when(r == 0)
    def _():
        acc_ref[...] = jnp.zeros_like(acc_ref)

    u = u_ref[...]
    for half in range(NC):
        sl = slice(half * HH, (half + 1) * HH)
        acc_ref[0:1, sl] += jnp.sum(u[0, half], axis=0, keepdims=True)
        acc_ref[1:2, sl] += jnp.sum(u[0, half] * u[0, half], axis=0,
                                    keepdims=True)
        acc_ref[2:3, sl] += jnp.sum(u[1, half], axis=0, keepdims=True)
        acc_ref[3:4, sl] += jnp.sum(u[1, half] * u[1, half], axis=0,
                                    keepdims=True)

    @pl.when(r == GRID - 1)
    def _():
        inv_n = 1.0 / N
        for (row0, g_ref, b_ref, out_ref) in ((0, ge_ref, be_ref, ste_ref),
                                              (2, gs_ref, bs_ref, sts_ref)):
            m = acc_ref[row0:row0 + 1, :] * inv_n
            v = acc_ref[row0 + 1:row0 + 2, :] * inv_n - m * m
            a = g_ref[...] * lax.rsqrt(v + 1e-5)
            b = b_ref[...] - a * m
            out_ref[...] = jnp.concatenate(
                [a, b, jnp.zeros((6, NHID), jnp.float32)], axis=0)


def _stats2(u, ge, be, gs, bs):
    return pl.pallas_call(
        _stats2_body,
        grid=(GRID,),
        in_specs=[
            pl.BlockSpec((2, NC, RB, HH), lambda r: (0, 0, r, 0)),
            pl.BlockSpec((1, NHID), lambda r: (0, 0)),
            pl.BlockSpec((1, NHID), lambda r: (0, 0)),
            pl.BlockSpec((1, NHID), lambda r: (0, 0)),
            pl.BlockSpec((1, NHID), lambda r: (0, 0)),
        ],
        out_specs=[
            pl.BlockSpec((8, NHID), lambda r: (0, 0)),
            pl.BlockSpec((8, NHID), lambda r: (0, 0)),
        ],
        out_shape=[
            jax.ShapeDtypeStruct((8, NHID), jnp.float32),
            jax.ShapeDtypeStruct((8, NHID), jnp.float32),
        ],
        scratch_shapes=[pltpu.VMEM((8, NHID), jnp.float32)],
    )(u, ge, be, gs, bs)


def _act_half(u, st, half):
    sl = slice(half * HH, (half + 1) * HH)
    hn = u * st[0:1, sl] + st[1:2, sl]
    return jnp.where(hn > 0, hn, jnp.exp(hn) - 1.0)


def _apply2_body(u_ref, ste_ref, sts_ref, we_ref, ws_ref, o_ref):
    u = u_ref[...]
    for (br, st_ref, w_ref) in ((0, ste_ref, we_ref), (1, sts_ref, ws_ref)):
        st = st_ref[...]
        y = jnp.dot(_act_half(u[br, 0], st, 0), w_ref[0:HH, :],
                    preferred_element_type=jnp.float32)
        y += jnp.dot(_act_half(u[br, 1], st, 1), w_ref[HH:2 * HH, :],
                     preferred_element_type=jnp.float32)
        o_ref[br, 0] = y[:, 0:HH]
        o_ref[br, 1] = y[:, HH:2 * HH]


def _apply2(u, ste, sts, we, ws):
    return pl.pallas_call(
        _apply2_body,
        grid=(GRID,),
        in_specs=[
            pl.BlockSpec((2, NC, RB, HH), lambda r: (0, 0, r, 0)),
            pl.BlockSpec((8, NHID), lambda r: (0, 0)),
            pl.BlockSpec((8, NHID), lambda r: (0, 0)),
            pl.BlockSpec((NHID, NHID), lambda r: (0, 0)),
            pl.BlockSpec((NHID, NHID), lambda r: (0, 0)),
        ],
        out_specs=pl.BlockSpec((2, NC, RB, HH), lambda r: (0, 0, r, 0)),
        out_shape=jax.ShapeDtypeStruct((2, NC, N, HH), jnp.float32),
    )(u, ste, sts, we, ws)


def _concat_body(u_ref, ste_ref, sts_ref, w_ref, o_ref):
    ste = ste_ref[...]
    sts = sts_ref[...]
    u = u_ref[...]
    w = w_ref[...]
    y = jnp.dot(_act_half(u[0, 0], ste, 0), w[0:HH, :],
                preferred_element_type=jnp.float32)
    y += jnp.dot(_act_half(u[0, 1], ste, 1), w[HH:2 * HH, :],
                 preferred_element_type=jnp.float32)
    y += jnp.dot(_act_half(u[1, 0], sts, 0), w[2 * HH:3 * HH, :],
                 preferred_element_type=jnp.float32)
    y += jnp.dot(_act_half(u[1, 1], sts, 1), w[3 * HH:4 * HH, :],
                 preferred_element_type=jnp.float32)
    o_ref[...] = y


def _concat_mm(u, ste, sts, w):
    return pl.pallas_call(
        _concat_body,
        grid=(GRID,),
        in_specs=[
            pl.BlockSpec((2, NC, RB, HH), lambda r: (0, 0, r, 0)),
            pl.BlockSpec((8, NHID), lambda r: (0, 0)),
            pl.BlockSpec((8, NHID), lambda r: (0, 0)),
            pl.BlockSpec((2 * NHID, NHID), lambda r: (0, 0)),
        ],
        out_specs=pl.BlockSpec((RB, NHID), lambda r: (r, 0)),
        out_shape=jax.ShapeDtypeStruct((N, NHID), jnp.float32),
    )(u, ste, sts, w)


def _statsf_body(h_ref, g_ref, b_ref, st_ref, acc_ref):
    r = pl.program_id(0)

    @pl.when(r == 0)
    def _():
        acc_ref[...] = jnp.zeros_like(acc_ref)

    h = h_ref[...]
    acc_ref[0:1, :] += jnp.sum(h, axis=0, keepdims=True)
    acc_ref[1:2, :] += jnp.sum(h * h, axis=0, keepdims=True)

    @pl.when(r == GRID - 1)
    def _():
        inv_n = 1.0 / N
        m = acc_ref[0:1, :] * inv_n
        v = acc_ref[1:2, :] * inv_n - m * m
        a = g_ref[...] * lax.rsqrt(v + 1e-5)
        b = b_ref[...] - a * m
        st_ref[...] = jnp.concatenate(
            [a, b, jnp.zeros((6, NHID), jnp.float32)], axis=0)


def _statsf(h, g, b):
    return pl.pallas_call(
        _statsf_body,
        grid=(GRID,),
        in_specs=[
            pl.BlockSpec((RB, NHID), lambda r: (r, 0)),
            pl.BlockSpec((1, NHID), lambda r: (0, 0)),
            pl.BlockSpec((1, NHID), lambda r: (0, 0)),
        ],
        out_specs=pl.BlockSpec((8, NHID), lambda r: (0, 0)),
        out_shape=jax.ShapeDtypeStruct((8, NHID), jnp.float32),
        scratch_shapes=[pltpu.VMEM((8, NHID), jnp.float32)],
    )(h, g, b)


def _applyf_body(h_ref, st_ref, w_ref, o_ref):
    st = st_ref[...]
    hn = h_ref[...] * st[0:1, :] + st[1:2, :]
    hact = jnp.where(hn > 0, hn, jnp.exp(hn) - 1.0)
    o_ref[...] = jnp.dot(hact, w_ref[...], preferred_element_type=jnp.float32)


def _applyf(h, st, w):
    return pl.pallas_call(
        _applyf_body,
        grid=(GRID,),
        in_specs=[
            pl.BlockSpec((RB, NHID), lambda r: (r, 0)),
            pl.BlockSpec((8, NHID), lambda r: (0, 0)),
            pl.BlockSpec((NHID, NHID), lambda r: (0, 0)),
        ],
        out_specs=pl.BlockSpec((RB, NHID), lambda r: (r, 0)),
        out_shape=jax.ShapeDtypeStruct((N, NHID), jnp.float32),
    )(h, st, w)


def _final_body(h_ref, st_ref, w_ref, bias_ref, o_ref):
    st = st_ref[...]
    hn = h_ref[...] * st[0:1, :] + st[1:2, :]
    hact = jnp.where(hn > 0, hn, jnp.exp(hn) - 1.0)
    z = jnp.dot(hact, w_ref[...], preferred_element_type=jnp.float32)
    z += bias_ref[...]
    m = jnp.max(z, axis=1, keepdims=True)
    zs = z - m
    lse = jnp.log(jnp.sum(jnp.exp(zs), axis=1, keepdims=True))
    o_ref[...] = zs - lse


def _final(h, st, w, bias):
    return pl.pallas_call(
        _final_body,
        grid=(GRID,),
        in_specs=[
            pl.BlockSpec((RB, NHID), lambda r: (r, 0)),
            pl.BlockSpec((8, NHID), lambda r: (0, 0)),
            pl.BlockSpec((NHID, NOUT), lambda r: (0, 0)),
            pl.BlockSpec((1, NOUT), lambda r: (0, 0)),
        ],
        out_specs=pl.BlockSpec((RB, NOUT), lambda r: (r, 0)),
        out_shape=jax.ShapeDtypeStruct((N, NOUT), jnp.float32),
    )(h, st, w, bias)


def kernel(x, edge_index_exp, edge_weight_exp, edge_index_sp, edge_weight_sp,
           params):
    p = params
    row_e, col_e = edge_index_exp[0], edge_index_exp[1]
    row_s, col_s = edge_index_sp[0], edge_index_sp[1]
    zeros_hbm = jnp.zeros((N, HH), jnp.float32)

    def spmm2(sup):
        return _spmm_call(col_e, row_e, edge_weight_exp,
                          col_s, row_s, edge_weight_sp, sup, zeros_hbm)

    def r2(v):
        return v.reshape(1, -1)

    wcat = jnp.concatenate([p['W_in_exp'], p['W_in_sp']], axis=1)
    sc = _inproj(x, wcat)

    for i in (0, 1, 2):
        sfx = ('in', 'c1', 'c2')[i]
        u = spmm2(sc)
        ste, sts = _stats2(u,
                           r2(p['g_%s_exp' % sfx]), r2(p['beta_%s_exp' % sfx]),
                           r2(p['g_%s_sp' % sfx]), r2(p['beta_%s_sp' % sfx]))
        if i < 2:
            nxt = ('c1', 'c2')[i]
            sc = _apply2(u, ste, sts,
                         p['W_%s_exp' % nxt], p['W_%s_sp' % nxt])
        else:
            o1 = _concat_mm(u, ste, sts, p['W_out11'])

    st1 = _statsf(o1, r2(p['g_out1']), r2(p['beta_out1']))
    o2 = _applyf(o1, st1, p['W_out111'])
    st2 = _statsf(o2, r2(p['g_out111']), r2(p['beta_out111']))
    return _final(o2, st2, p['W_out12'], r2(p['b_out12']))


# speedup vs baseline: 1.3451x; 1.0258x over previous
import functools

import jax
import jax.numpy as jnp
from jax import lax
from jax.experimental import pallas as pl
from jax.experimental.pallas import tpu as pltpu
from jax.experimental.pallas import tpu_sc as plsc

N = 10000
E = 320000
NFEAT = 128
NHID = 256
HH = 128
NOUT = 32

NC = 2
NS = 16
EPW = E // NS
K = 80
NB = EPW // K
SLAB = 624
REM = N - NS * SLAB

RB = 1000
GRID = N // RB


NBUF = 3
CH = 10
NCH = NB // CH
MAIN = (NB - 1) // NBUF * NBUF


def _spmm_body(pk_h, w_h, sup_h, z_h, out_h, pk_v, w_v, bufs, gsems,
               ssems, pksem, wsem, acc_s):
    c = lax.axis_index("c")
    s = lax.axis_index("s")

    soff = pl.multiple_of(s * SLAB, 8)
    pltpu.sync_copy(z_h.at[pl.ds(soff, SLAB)], acc_s.at[pl.ds(soff, SLAB)])

    @pl.when(s == 0)
    def _():
        pltpu.sync_copy(z_h.at[pl.ds(NS * SLAB, REM)],
                        acc_s.at[pl.ds(NS * SLAB, REM)])

    pltpu.sync_copy(pk_h.at[c, s, 0], pk_v.at[0])
    pltpu.sync_copy(w_h.at[s, 0], w_v.at[0])
    plsc.subcore_barrier()

    def gather(b, j):
        return pltpu.async_copy(
            sup_h.at[pk_v.at[(b // CH) % 2, b % CH, 0]], bufs[j], gsems[j])

    def wait_gather(b, j):
        pltpu.make_async_copy(
            sup_h.at[pk_v.at[(b // CH) % 2, b % CH, 0]], bufs[j],
            gsems[j]).wait()

    def scatter(b, j):
        return pltpu.async_copy(
            bufs[j], acc_s.at[pk_v.at[(b // CH) % 2, b % CH, 1]], ssems[j],
            add=True)

    def wait_scatter(b, j):
        pltpu.make_async_copy(
            bufs[j], acc_s.at[pk_v.at[(b // CH) % 2, b % CH, 1]],
            ssems[j]).wait()

    def scale(b, j):
        d = (b // CH) % 2
        bb = b % CH

        def group(g, carry2):
            goff = pl.multiple_of(g * 16, 16)
            w16 = w_v[d, bb, pl.ds(goff, 16)]
            for u in range(16):
                we = w16[u]
                e = goff + u
                for k in range(HH // 16):
                    bufs[j][e, pl.ds(k * 16, 16)] = (
                        bufs[j][e, pl.ds(k * 16, 16)] * we)
            return carry2
        lax.fori_loop(0, K // 16, group, 0)

    gather(0, 0)
    gather(1, 1)

    def step(o, carry):
        for j in range(NBUF):
            b = o * NBUF + j
            wait_gather(b, j)
            scale(b, j)
            scatter(b, j)

            bn = b + 2
            jn = (j + 2) % NBUF
            n = b // CH
            bb = b % CH

            @pl.when(jnp.logical_and(bb == CH - 2, n < NCH - 1))
            def _():
                pltpu.make_async_copy(
                    pk_h.at[c, s, n + 1],
                    pk_v.at[(n + 1) % 2], pksem).wait()
                pltpu.make_async_copy(
                    w_h.at[s, n + 1],
                    w_v.at[(n + 1) % 2], wsem).wait()

            @pl.when(bn < NB)
            def _():
                @pl.when(bn >= NBUF)
                def _():
                    wait_scatter(bn - NBUF, jn)
                gather(bn, jn)

            @pl.when(jnp.logical_and(bb == 0, n + 1 < NCH))
            def _():
                pltpu.async_copy(pk_h.at[c, s, n + 1],
                                 pk_v.at[(n + 1) % 2], pksem)
                pltpu.async_copy(w_h.at[s, n + 1],
                                 w_v.at[(n + 1) % 2], wsem)
        return carry

    lax.fori_loop(0, MAIN // NBUF, step, 0)

    for b in range(MAIN, NB):
        j = b % NBUF
        wait_gather(b, j)
        scale(b, j)
        scatter(b, j)

    for b in range(NB - NBUF, NB):
        wait_scatter(b, b % NBUF)

    plsc.subcore_barrier()
    pl.delay(2000)
    plsc.subcore_barrier()

    pltpu.sync_copy(acc_s.at[pl.ds(soff, SLAB)],
                    out_h.at[c, pl.ds(soff, SLAB)])

    @pl.when(s == 0)
    def _():
        pltpu.sync_copy(acc_s.at[pl.ds(NS * SLAB, REM)],
                        out_h.at[c, pl.ds(NS * SLAB, REM)])


@functools.cache
def _spmm_kernel():
    return pl.kernel(
        _spmm_body,
        out_type=jax.ShapeDtypeStruct((NC, N, HH), jnp.float32),
        mesh=plsc.VectorSubcoreMesh(core_axis_name="c", subcore_axis_name="s",
                                    num_cores=NC, num_subcores=NS),
        scratch_types=[
            pltpu.VMEM((2, CH, 2, K), jnp.int32),
            pltpu.VMEM((2, CH, K), jnp.float32),
            tuple(pltpu.VMEM((K, HH), jnp.float32) for _ in range(NBUF)),
            tuple(pltpu.SemaphoreType.DMA for _ in range(NBUF)),
            tuple(pltpu.SemaphoreType.DMA for _ in range(NBUF)),
            pltpu.SemaphoreType.DMA,
            pltpu.SemaphoreType.DMA,
            pltpu.VMEM_SHARED((N, HH), jnp.float32),
        ],
    )


def _spmm_call(col, row, w, sup, zeros):
    col2 = jnp.stack([col, col + N]).reshape(NC, NS, NCH, CH, 1, K)
    row4 = jnp.broadcast_to(row.reshape(1, NS, NCH, CH, 1, K),
                            (NC, NS, NCH, CH, 1, K))
    pk = jnp.concatenate([col2, row4], axis=4)
    w3 = w.reshape(NS, NCH, CH, K)
    return _spmm_kernel()(pk, w3, sup, zeros)


def _inproj_body(x_ref, w_ref, oe_ref, os_ref):
    y = jnp.dot(x_ref[...], w_ref[...], preferred_element_type=jnp.float32)
    oe_ref[0] = y[:, 0:HH]
    oe_ref[1] = y[:, HH:2 * HH]
    os_ref[0] = y[:, 2 * HH:3 * HH]
    os_ref[1] = y[:, 3 * HH:4 * HH]


def _inproj(x, wcat):
    return pl.pallas_call(
        _inproj_body,
        grid=(GRID,),
        in_specs=[
            pl.BlockSpec((RB, NFEAT), lambda r: (r, 0)),
            pl.BlockSpec((NFEAT, 2 * NHID), lambda r: (0, 0)),
        ],
        out_specs=[
            pl.BlockSpec((NC, RB, HH), lambda r: (0, r, 0)),
            pl.BlockSpec((NC, RB, HH), lambda r: (0, r, 0)),
        ],
        out_shape=[
            jax.ShapeDtypeStruct((NC, N, HH), jnp.float32),
            jax.ShapeDtypeStruct((NC, N, HH), jnp.float32),
        ],
    )(x, wcat)


def _stats2_body(ue_ref, us_ref, ge_ref, be_ref, gs_ref, bs_ref,
                 ste_ref, sts_ref, acc_ref):
    r = pl.program_id(0)

    @pl.when(r == 0)
    def _():
        acc_ref[...] = jnp.zeros_like(acc_ref)

    ue = ue_ref[...]
    us = us_ref[...]
    for half in range(NC):
        sl = slice(half * HH, (half + 1) * HH)
        acc_ref[0:1, sl] += jnp.sum(ue[half], axis=0, keepdims=True)
        acc_ref[1:2, sl] += jnp.sum(ue[half] * ue[half], axis=0, keepdims=True)
        acc_ref[2:3, sl] += jnp.sum(us[half], axis=0, keepdims=True)
        acc_ref[3:4, sl] += jnp.sum(us[half] * us[half], axis=0, keepdims=True)

    @pl.when(r == GRID - 1)
    def _():
        inv_n = 1.0 / N
        for (row0, g_ref, b_ref, out_ref) in ((0, ge_ref, be_ref, ste_ref),
                                              (2, gs_ref, bs_ref, sts_ref)):
            m = acc_ref[row0:row0 + 1, :] * inv_n
            v = acc_ref[row0 + 1:row0 + 2, :] * inv_n - m * m
            a = g_ref[...] * lax.rsqrt(v + 1e-5)
            b = b_ref[...] - a * m
            out_ref[...] = jnp.concatenate(
                [a, b, jnp.zeros((6, NHID), jnp.float32)], axis=0)


def _stats2(ue, us, ge, be, gs, bs):
    return pl.pallas_call(
        _stats2_body,
        grid=(GRID,),
        in_specs=[
            pl.BlockSpec((NC, RB, HH), lambda r: (0, r, 0)),
            pl.BlockSpec((NC, RB, HH), lambda r: (0, r, 0)),
            pl.BlockSpec((1, NHID), lambda r: (0, 0)),
            pl.BlockSpec((1, NHID), lambda r: (0, 0)),
            pl.BlockSpec((1, NHID), lambda r: (0, 0)),
            pl.BlockSpec((1, NHID), lambda r: (0, 0)),
        ],
        out_specs=[
            pl.BlockSpec((8, NHID), lambda r: (0, 0)),
            pl.BlockSpec((8, NHID), lambda r: (0, 0)),
        ],
        out_shape=[
            jax.ShapeDtypeStruct((8, NHID), jnp.float32),
            jax.ShapeDtypeStruct((8, NHID), jnp.float32),
        ],
        scratch_shapes=[pltpu.VMEM((8, NHID), jnp.float32)],
    )(ue, us, ge, be, gs, bs)


def _act_half(u, st, half):
    sl = slice(half * HH, (half + 1) * HH)
    hn = u * st[0:1, sl] + st[1:2, sl]
    return jnp.where(hn > 0, hn, jnp.exp(hn) - 1.0)


def _apply2_body(ue_ref, us_ref, ste_ref, sts_ref, we_ref, ws_ref,
                 oe_ref, os_ref):
    ste = ste_ref[...]
    sts = sts_ref[...]
    for (u_ref, st, w_ref, o_ref) in ((ue_ref, ste, we_ref, oe_ref),
                                      (us_ref, sts, ws_ref, os_ref)):
        u = u_ref[...]
        y = jnp.dot(_act_half(u[0], st, 0), w_ref[0:HH, :],
                    preferred_element_type=jnp.float32)
        y += jnp.dot(_act_half(u[1], st, 1), w_ref[HH:2 * HH, :],
                     preferred_element_type=jnp.float32)
        o_ref[0] = y[:, 0:HH]
        o_ref[1] = y[:, HH:2 * HH]


def _apply2(ue, us, ste, sts, we, ws):
    return pl.pallas_call(
        _apply2_body,
        grid=(GRID,),
        in_specs=[
            pl.BlockSpec((NC, RB, HH), lambda r: (0, r, 0)),
            pl.BlockSpec((NC, RB, HH), lambda r: (0, r, 0)),
            pl.BlockSpec((8, NHID), lambda r: (0, 0)),
            pl.BlockSpec((8, NHID), lambda r: (0, 0)),
            pl.BlockSpec((NHID, NHID), lambda r: (0, 0)),
            pl.BlockSpec((NHID, NHID), lambda r: (0, 0)),
        ],
        out_specs=[
            pl.BlockSpec((NC, RB, HH), lambda r: (0, r, 0)),
            pl.BlockSpec((NC, RB, HH), lambda r: (0, r, 0)),
        ],
        out_shape=[
            jax.ShapeDtypeStruct((NC, N, HH), jnp.float32),
            jax.ShapeDtypeStruct((NC, N, HH), jnp.float32),
        ],
    )(ue, us, ste, sts, we, ws)


def _concat_body(ue_ref, us_ref, ste_ref, sts_ref, w_ref, o_ref):
    ste = ste_ref[...]
    sts = sts_ref[...]
    ue = ue_ref[...]
    us = us_ref[...]
    w = w_ref[...]
    y = jnp.dot(_act_half(ue[0], ste, 0), w[0:HH, :],
                preferred_element_type=jnp.float32)
    y += jnp.dot(_act_half(ue[1], ste, 1), w[HH:2 * HH, :],
                 preferred_element_type=jnp.float32)
    y += jnp.dot(_act_half(us[0], sts, 0), w[2 * HH:3 * HH, :],
                 preferred_element_type=jnp.float32)
    y += jnp.dot(_act_half(us[1], sts, 1), w[3 * HH:4 * HH, :],
                 preferred_element_type=jnp.float32)
    o_ref[...] = y


def _concat_mm(ue, us, ste, sts, w):
    return pl.pallas_call(
        _concat_body,
        grid=(GRID,),
        in_specs=[
            pl.BlockSpec((NC, RB, HH), lambda r: (0, r, 0)),
            pl.BlockSpec((NC, RB, HH), lambda r: (0, r, 0)),
            pl.BlockSpec((8, NHID), lambda r: (0, 0)),
            pl.BlockSpec((8, NHID), lambda r: (0, 0)),
            pl.BlockSpec((2 * NHID, NHID), lambda r: (0, 0)),
        ],
        out_specs=pl.BlockSpec((RB, NHID), lambda r: (r, 0)),
        out_shape=jax.ShapeDtypeStruct((N, NHID), jnp.float32),
    )(ue, us, ste, sts, w)


def _statsf_body(h_ref, g_ref, b_ref, st_ref, acc_ref):
    r = pl.program_id(0)

    @pl.when(r == 0)
    def _():
        acc_ref[...] = jnp.zeros_like(acc_ref)

    h = h_ref[...]
    acc_ref[0:1, :] += jnp.sum(h, axis=0, keepdims=True)
    acc_ref[1:2, :] += jnp.sum(h * h, axis=0, keepdims=True)

    @pl.when(r == GRID - 1)
    def _():
        inv_n = 1.0 / N
        m = acc_ref[0:1, :] * inv_n
        v = acc_ref[1:2, :] * inv_n - m * m
        a = g_ref[...] * lax.rsqrt(v + 1e-5)
        b = b_ref[...] - a * m
        st_ref[...] = jnp.concatenate(
            [a, b, jnp.zeros((6, NHID), jnp.float32)], axis=0)


def _statsf(h, g, b):
    return pl.pallas_call(
        _statsf_body,
        grid=(GRID,),
        in_specs=[
            pl.BlockSpec((RB, NHID), lambda r: (r, 0)),
            pl.BlockSpec((1, NHID), lambda r: (0, 0)),
            pl.BlockSpec((1, NHID), lambda r: (0, 0)),
        ],
        out_specs=pl.BlockSpec((8, NHID), lambda r: (0, 0)),
        out_shape=jax.ShapeDtypeStruct((8, NHID), jnp.float32),
        scratch_shapes=[pltpu.VMEM((8, NHID), jnp.float32)],
    )(h, g, b)


def _applyf_body(h_ref, st_ref, w_ref, o_ref):
    st = st_ref[...]
    hn = h_ref[...] * st[0:1, :] + st[1:2, :]
    hact = jnp.where(hn > 0, hn, jnp.exp(hn) - 1.0)
    o_ref[...] = jnp.dot(hact, w_ref[...], preferred_element_type=jnp.float32)


def _applyf(h, st, w):
    return pl.pallas_call(
        _applyf_body,
        grid=(GRID,),
        in_specs=[
            pl.BlockSpec((RB, NHID), lambda r: (r, 0)),
            pl.BlockSpec((8, NHID), lambda r: (0, 0)),
            pl.BlockSpec((NHID, NHID), lambda r: (0, 0)),
        ],
        out_specs=pl.BlockSpec((RB, NHID), lambda r: (r, 0)),
        out_shape=jax.ShapeDtypeStruct((N, NHID), jnp.float32),
    )(h, st, w)


def _final_body(h_ref, st_ref, w_ref, bias_ref, o_ref):
    st = st_ref[...]
    hn = h_ref[...] * st[0:1, :] + st[1:2, :]
    hact = jnp.where(hn > 0, hn, jnp.exp(hn) - 1.0)
    z = jnp.dot(hact, w_ref[...], preferred_element_type=jnp.float32)
    z += bias_ref[...]
    m = jnp.max(z, axis=1, keepdims=True)
    zs = z - m
    lse = jnp.log(jnp.sum(jnp.exp(zs), axis=1, keepdims=True))
    o_ref[...] = zs - lse


def _final(h, st, w, bias):
    return pl.pallas_call(
        _final_body,
        grid=(GRID,),
        in_specs=[
            pl.BlockSpec((RB, NHID), lambda r: (r, 0)),
            pl.BlockSpec((8, NHID), lambda r: (0, 0)),
            pl.BlockSpec((NHID, NOUT), lambda r: (0, 0)),
            pl.BlockSpec((1, NOUT), lambda r: (0, 0)),
        ],
        out_specs=pl.BlockSpec((RB, NOUT), lambda r: (r, 0)),
        out_shape=jax.ShapeDtypeStruct((N, NOUT), jnp.float32),
    )(h, st, w, bias)


def kernel(x, edge_index_exp, edge_weight_exp, edge_index_sp, edge_weight_sp,
           params):
    p = params
    row_e, col_e = edge_index_exp[0], edge_index_exp[1]
    row_s, col_s = edge_index_sp[0], edge_index_sp[1]
    zeros_hbm = jnp.zeros((N, HH), jnp.float32)

    def spmm(row, col, w, sup):
        return _spmm_call(col, row, w, sup.reshape(NC * N, HH), zeros_hbm)

    def r2(v):
        return v.reshape(1, -1)

    wcat = jnp.concatenate([p['W_in_exp'], p['W_in_sp']], axis=1)
    se, ss = _inproj(x, wcat)

    for i in (0, 1, 2):
        sfx = ('in', 'c1', 'c2')[i]
        ue = spmm(row_e, col_e, edge_weight_exp, se)
        us = spmm(row_s, col_s, edge_weight_sp, ss)
        ste, sts = _stats2(ue, us,
                           r2(p['g_%s_exp' % sfx]), r2(p['beta_%s_exp' % sfx]),
                           r2(p['g_%s_sp' % sfx]), r2(p['beta_%s_sp' % sfx]))
        if i < 2:
            nxt = ('c1', 'c2')[i]
            se, ss = _apply2(ue, us, ste, sts,
                             p['W_%s_exp' % nxt], p['W_%s_sp' % nxt])
        else:
            o1 = _concat_mm(ue, us, ste, sts, p['W_out11'])

    st1 = _statsf(o1, r2(p['g_out1']), r2(p['beta_out1']))
    o2 = _applyf(o1, st1, p['W_out111'])
    st2 = _statsf(o2, r2(p['g_out111']), r2(p['beta_out111']))
    return _final(o2, st2, p['W_out12'], r2(p['b_out12']))
